# baseline (speedup 1.0000x reference)
import functools

import jax
import jax.numpy as jnp
from jax import lax
from jax.experimental import pallas as pl
from jax.experimental.pallas import tpu as pltpu
from jax.experimental.pallas import tpu_sc as plsc

N_FEAT = 128
N_GATES = 64
MAX_CONN = N_FEAT + 2 + N_GATES
CHUNKS = 13
GW_WORDS = N_GATES * MAX_CONN
SUB = 8
LANES = 512
BLK = SUB * LANES


def _topk_sc_body(nc, gpw, gw_hbm, idx_hbm, gw_v, prob_v, out_v):
    wid = lax.axis_index("s") * nc + lax.axis_index("c")
    big = jnp.int32(1 << 30)
    lane = lax.iota(jnp.int32, 16)

    pltpu.sync_copy(gw_hbm, gw_v.at[pl.ds(0, GW_WORDS)])

    for j in range(gpw):
        g = wid * gpw + j
        n_valid = N_FEAT + 2 + g
        base = g * MAX_CONN

        def chunk(c):
            x = gw_v[pl.ds(base + c * 16, 16)]
            col = lane + c * 16
            return jnp.where(col < n_valid, x, -1e30), col

        m = jnp.float32(-1e30)
        for c in range(CHUNKS):
            x, _ = chunk(c)
            m = jnp.maximum(m, jnp.max(x))

        s = jnp.float32(0.0)
        for c in range(CHUNKS):
            x, col = chunk(c)
            e = jnp.where(col < n_valid, jnp.exp(x - m), 0.0)
            prob_v[pl.ds(c * 16, 16)] = e
            s = s + jnp.sum(e)

        m1 = jnp.float32(-1.0)
        for c in range(CHUNKS):
            p = prob_v[pl.ds(c * 16, 16)] / s
            prob_v[pl.ds(c * 16, 16)] = p
            m1 = jnp.maximum(m1, jnp.max(p))

        i1 = big
        for c in range(CHUNKS):
            p = prob_v[pl.ds(c * 16, 16)]
            i1 = jnp.minimum(i1, jnp.min(jnp.where(p == m1, lane + c * 16, big)))

        m2 = jnp.float32(-1.0)
        for c in range(CHUNKS):
            p = prob_v[pl.ds(c * 16, 16)]
            p = jnp.where(lane + c * 16 == i1, -1.0, p)
            m2 = jnp.maximum(m2, jnp.max(p))

        i2 = big
        for c in range(CHUNKS):
            p = prob_v[pl.ds(c * 16, 16)]
            p = jnp.where(lane + c * 16 == i1, -1.0, p)
            i2 = jnp.minimum(i2, jnp.min(jnp.where(p == m2, lane + c * 16, big)))

        out_v[...] = jnp.where(lane == 0, i1, jnp.where(lane == 1, i2, 0))
        pltpu.sync_copy(out_v, idx_hbm.at[g])


def _fused_chain_kernel(n_out, idx_ref, w_ref, scale_ref, x_ref, out_ref,
                        avail_ref):
    for j in range(SUB):
        avail_ref[0:N_FEAT, j] = x_ref[j].T
    avail_ref[N_FEAT] = jnp.zeros((SUB, LANES), jnp.float32)
    avail_ref[N_FEAT + 1] = jnp.ones((SUB, LANES), jnp.float32)

    acc = [jnp.zeros((SUB, LANES), jnp.float32) for _ in range(n_out)]
    for g in range(N_GATES):
        ia = idx_ref[g, 0]
        ib = idx_ref[g, 1]
        row = 1.0 - avail_ref[ia] * avail_ref[ib]
        avail_ref[N_FEAT + 2 + g] = row
        for o in range(n_out):
            acc[o] = acc[o] + w_ref[g, o] * row
    for o in range(n_out):
        out_ref[o] = acc[o] * scale_ref[o]


def kernel(X, gate_weights, output_weights, output_scale):
    n = X.shape[0]
    n_out = output_weights.shape[1]
    nblk = n // BLK

    info = plsc.get_sparse_core_info()
    nc, ns = info.num_cores, info.num_subcores
    gpw = N_GATES // (nc * ns)

    topk = functools.partial(
        pl.kernel,
        mesh=plsc.VectorSubcoreMesh(core_axis_name="c", subcore_axis_name="s"),
        compiler_params=pltpu.CompilerParams(needs_layout_passes=False),
        out_type=jax.ShapeDtypeStruct((N_GATES, 16), jnp.int32),
        scratch_types=[
            pltpu.VMEM((GW_WORDS + 16, ), jnp.float32),
            pltpu.VMEM((CHUNKS * 16,), jnp.float32),
            pltpu.VMEM((16,), jnp.int32),
        ],
    )(functools.partial(_topk_sc_body, nc, gpw))
    idx = topk(gate_weights.reshape(GW_WORDS))

    x3 = X.reshape(n // LANES, LANES, N_FEAT)
    out3 = pl.pallas_call(
        functools.partial(_fused_chain_kernel, n_out),
        grid=(nblk,),
        in_specs=[
            pl.BlockSpec(memory_space=pltpu.SMEM),
            pl.BlockSpec(memory_space=pltpu.SMEM),
            pl.BlockSpec(memory_space=pltpu.SMEM),
            pl.BlockSpec((SUB, LANES, N_FEAT), lambda i: (i, 0, 0)),
        ],
        out_specs=pl.BlockSpec((n_out, SUB, LANES), lambda i: (0, i, 0)),
        out_shape=jax.ShapeDtypeStruct((n_out, n // LANES, LANES), jnp.float32),
        scratch_shapes=[pltpu.VMEM((MAX_CONN, SUB, LANES), jnp.float32)],
    )(idx, output_weights, output_scale, x3)
    return out3.reshape(n_out, n).T
